# Initial kernel scaffold; baseline (speedup 1.0000x reference)
import jax
import jax.numpy as jnp
from jax.experimental import pallas as pl


def kernel(x, table):
    raise NotImplementedError("write your pallas kernel here")


# speedup vs baseline: 6.7436x; 6.7436x over previous
import functools

import jax
import jax.numpy as jnp
from jax import lax
from jax.experimental import pallas as pl
from jax.experimental.pallas import tpu as pltpu
from jax.experimental.pallas import tpu_sc as plsc

EMBED_DIM = 64
BATCH = 4096
T = 30
NF = 6
OUT_D = (NF - 1) * EMBED_DIM

NC = 2
NS = 16
NW = NC * NS
PAIRS_PER_W = BATCH // (2 * NW)
IDX_PER_PAIR = 2 * T * NF
GATHER_CHUNK = 120
NCHUNK = IDX_PER_PAIR // GATHER_CHUNK


def _rsqrt16(s):
    i = lax.bitcast_convert_type(s, jnp.int32)
    y = lax.bitcast_convert_type(jnp.int32(0x5F3759DF) - (i >> 1), jnp.float32)
    for _ in range(3):
        y = y * (jnp.float32(1.5) - jnp.float32(0.5) * s * y * y)
    return y


def _body(x_hbm, table_hbm, out_hbm, idx_v, rows_v, out_v, sem):
    wid = lax.axis_index("s") * NC + lax.axis_index("c")

    def pair_body(p, carry):
        g = wid * PAIRS_PER_W + p
        for c in range(NCHUNK):
            pltpu.sync_copy(
                x_hbm.at[pl.ds(g * IDX_PER_PAIR + c * GATHER_CHUNK, GATHER_CHUNK)],
                idx_v.at[c],
            )
        cps = [
            pltpu.async_copy(
                table_hbm.at[idx_v.at[c]],
                rows_v.at[pl.ds(c * GATHER_CHUNK, GATHER_CHUNK)],
                sem,
            )
            for c in range(NCHUNK)
        ]
        for cp in cps:
            cp.wait()

        for be in range(2):
            r0 = be * (T * NF)

            def t_body(t, acc):
                base = r0 + t * NF
                f5 = [rows_v[base + 5, pl.ds(j * 16, 16)] for j in range(4)]
                new_acc = list(acc)
                for i in range(5):
                    for j in range(4):
                        v = rows_v[base + i, pl.ds(j * 16, 16)] + f5[j]
                        out_v[be, t, pl.ds(i * 64 + j * 16, 16)] = v
                        k = i * 4 + j
                        new_acc[k] = acc[k] + v * v
                return tuple(new_acc)

            zero = jnp.zeros((16,), jnp.float32)
            acc = lax.fori_loop(0, T, t_body, tuple(zero for _ in range(20)))

            scales = []
            for k in range(20):
                s = acc[k]
                y = _rsqrt16(s)
                scales.append(
                    jnp.where(s >= jnp.float32(1e-24), y, jnp.float32(1e12))
                )

            def scale_body(t, carry2):
                for i in range(5):
                    for j in range(4):
                        sl = pl.ds(i * 64 + j * 16, 16)
                        out_v[be, t, sl] = out_v[be, t, sl] * scales[i * 4 + j]
                return carry2

            lax.fori_loop(0, T, scale_body, 0)

        pltpu.sync_copy(out_v, out_hbm.at[pl.ds(g * 2, 2)])
        return carry

    lax.fori_loop(0, PAIRS_PER_W, pair_body, 0)


_sc_call = functools.partial(
    pl.kernel,
    out_type=jax.ShapeDtypeStruct((BATCH, T, OUT_D), jnp.float32),
    mesh=plsc.VectorSubcoreMesh(core_axis_name="c", subcore_axis_name="s"),
    compiler_params=pltpu.CompilerParams(use_tc_tiling_on_sc=False),
    scratch_types=[
        pltpu.VMEM((NCHUNK, GATHER_CHUNK), jnp.int32),
        pltpu.VMEM((IDX_PER_PAIR, EMBED_DIM), jnp.float32),
        pltpu.VMEM((2, T, OUT_D), jnp.float32),
        pltpu.SemaphoreType.DMA,
    ],
)(_body)


def kernel(x, table):
    return _sc_call(x.reshape(-1), table)


# speedup vs baseline: 8.5143x; 1.2626x over previous
import functools

import jax
import jax.numpy as jnp
from jax import lax
from jax.experimental import pallas as pl
from jax.experimental.pallas import tpu as pltpu
from jax.experimental.pallas import tpu_sc as plsc

EMBED_DIM = 64
BATCH = 4096
T = 30
NF = 6
OUT_D = (NF - 1) * EMBED_DIM

NC = 2
NS = 16
NW = NC * NS
PAIRS_PER_W = BATCH // (2 * NW)
IDX_PER_PAIR = 2 * T * NF
GATHER_CHUNK = 120
NCHUNK = IDX_PER_PAIR // GATHER_CHUNK
IDX_ROWS = PAIRS_PER_W * NCHUNK


def _rsqrt16(s):
    i = lax.bitcast_convert_type(s, jnp.int32)
    y = lax.bitcast_convert_type(jnp.int32(0x5F3759DF) - (i >> 1), jnp.float32)
    for _ in range(3):
        y = y * (jnp.float32(1.5) - jnp.float32(0.5) * s * y * y)
    return y


def _body(x_hbm, table_hbm, out_hbm, idx_v, rows_v, out_v,
          sem_g0, sem_g1, sem_o0, sem_o1):
    wid = lax.axis_index("s") * NC + lax.axis_index("c")
    sem_g = (sem_g0, sem_g1)
    sem_o = (sem_o0, sem_o1)

    pltpu.sync_copy(x_hbm.at[pl.ds(wid * IDX_ROWS, IDX_ROWS)], idx_v)

    def fire_gathers(p, buf):
        for c in range(NCHUNK):
            pltpu.async_copy(
                table_hbm.at[idx_v.at[p * NCHUNK + c]],
                rows_v.at[buf, pl.ds(c * GATHER_CHUNK, GATHER_CHUNK)],
                sem_g[buf],
            )

    def wait_gathers(buf):
        for c in range(NCHUNK):
            pltpu.make_async_copy(
                table_hbm.at[idx_v.at[c]],
                rows_v.at[buf, pl.ds(c * GATHER_CHUNK, GATHER_CHUNK)],
                sem_g[buf],
            ).wait()

    def wait_out(buf):
        pltpu.make_async_copy(
            out_v.at[buf], out_hbm.at[pl.ds(0, 2)], sem_o[buf]
        ).wait()

    def compute(g, buf):
        for be in range(2):
            r0 = be * (T * NF)

            def t_body(t, acc):
                base = r0 + t * NF
                f5 = [rows_v[buf, base + 5, pl.ds(j * 16, 16)]
                      for j in range(4)]
                new_acc = list(acc)
                for i in range(5):
                    for j in range(4):
                        v = rows_v[buf, base + i, pl.ds(j * 16, 16)] + f5[j]
                        out_v[buf, be, t, pl.ds(i * 64 + j * 16, 16)] = v
                        k = i * 4 + j
                        new_acc[k] = acc[k] + v * v
                return tuple(new_acc)

            zero = jnp.zeros((16,), jnp.float32)
            acc = lax.fori_loop(0, T, t_body, tuple(zero for _ in range(20)))

            scales = []
            for k in range(20):
                s = acc[k]
                y = _rsqrt16(s)
                scales.append(
                    jnp.where(s >= jnp.float32(1e-24), y, jnp.float32(1e12))
                )

            def scale_body(t, carry2):
                for i in range(5):
                    for j in range(4):
                        sl = pl.ds(i * 64 + j * 16, 16)
                        out_v[buf, be, t, sl] = (
                            out_v[buf, be, t, sl] * scales[i * 4 + j])
                return carry2

            lax.fori_loop(0, T, scale_body, 0)

        pltpu.async_copy(
            out_v.at[buf], out_hbm.at[pl.ds(g * 2, 2)], sem_o[buf])

    fire_gathers(0, 0)

    def step_body(s, carry):
        for buf in range(2):
            p = s * 2 + buf

            @pl.when(p + 1 < PAIRS_PER_W)
            def _():
                fire_gathers(p + 1, 1 - buf)

            wait_gathers(buf)

            @pl.when(p >= 2)
            def _():
                wait_out(buf)

            compute(wid * PAIRS_PER_W + p, buf)
        return carry

    lax.fori_loop(0, PAIRS_PER_W // 2, step_body, 0)
    wait_out(0)
    wait_out(1)


_sc_call = functools.partial(
    pl.kernel,
    out_type=jax.ShapeDtypeStruct((BATCH, T, OUT_D), jnp.float32),
    mesh=plsc.VectorSubcoreMesh(core_axis_name="c", subcore_axis_name="s"),
    compiler_params=pltpu.CompilerParams(use_tc_tiling_on_sc=False),
    scratch_types=[
        pltpu.VMEM((IDX_ROWS, GATHER_CHUNK), jnp.int32),
        pltpu.VMEM((2, IDX_PER_PAIR, EMBED_DIM), jnp.float32),
        pltpu.VMEM((2, 2, T, OUT_D), jnp.float32),
        pltpu.SemaphoreType.DMA,
        pltpu.SemaphoreType.DMA,
        pltpu.SemaphoreType.DMA,
        pltpu.SemaphoreType.DMA,
    ],
)(_body)


def kernel(x, table):
    x2 = x.reshape(BATCH * T * NF // GATHER_CHUNK, GATHER_CHUNK)
    return _sc_call(x2, table)


# speedup vs baseline: 8.5349x; 1.0024x over previous
import functools

import jax
import jax.numpy as jnp
from jax import lax
from jax.experimental import pallas as pl
from jax.experimental.pallas import tpu as pltpu
from jax.experimental.pallas import tpu_sc as plsc

EMBED_DIM = 64
BATCH = 4096
T = 30
NF = 6
OUT_D = (NF - 1) * EMBED_DIM

NC = 2
NS = 16
NW = NC * NS
PAIRS_PER_W = BATCH // (2 * NW)
IDX_PER_PAIR = 2 * T * NF


def _rsqrt16(s):
    i = lax.bitcast_convert_type(s, jnp.int32)
    y = lax.bitcast_convert_type(jnp.int32(0x5F3759DF) - (i >> 1), jnp.float32)
    for _ in range(3):
        y = y * (jnp.float32(1.5) - jnp.float32(0.5) * s * y * y)
    return y


def _body(x_hbm, table_hbm, out_hbm, idx_v, rows_v, out_v,
          sem_g0, sem_g1, sem_o0, sem_o1):
    wid = lax.axis_index("s") * NC + lax.axis_index("c")
    sem_g = (sem_g0, sem_g1)
    sem_o = (sem_o0, sem_o1)

    pltpu.sync_copy(x_hbm.at[pl.ds(wid * PAIRS_PER_W, PAIRS_PER_W)], idx_v)

    def fire_gathers(p, buf):
        pltpu.async_copy(
            table_hbm.at[idx_v.at[p]],
            rows_v.at[buf],
            sem_g[buf],
        )

    def wait_gathers(buf):
        pltpu.make_async_copy(
            table_hbm.at[idx_v.at[0]],
            rows_v.at[buf],
            sem_g[buf],
        ).wait()

    def wait_out(buf):
        pltpu.make_async_copy(
            out_v.at[buf], out_hbm.at[pl.ds(0, 2)], sem_o[buf]
        ).wait()

    def compute(g, buf):
        for be in range(2):
            def t_body(t, acc):
                base = (be * T + t) * NF
                f5 = [rows_v[buf, base + 5, pl.ds(j * 16, 16)]
                      for j in range(4)]
                new_acc = list(acc)
                for i in range(5):
                    for j in range(4):
                        v = (rows_v[buf, base + i, pl.ds(j * 16, 16)]
                             + f5[j])
                        out_v[buf, be, t, pl.ds(i * 64 + j * 16, 16)] = v
                        k = i * 4 + j
                        new_acc[k] = acc[k] + v * v
                return tuple(new_acc)

            zero = jnp.zeros((16,), jnp.float32)
            acc = lax.fori_loop(0, T, t_body, tuple(zero for _ in range(20)))

            scales = []
            for k in range(20):
                s = acc[k]
                y = _rsqrt16(s)
                scales.append(
                    jnp.where(s >= jnp.float32(1e-24), y, jnp.float32(1e12))
                )

            def scale_body(t, carry2):
                for i in range(5):
                    for j in range(4):
                        sl = pl.ds(i * 64 + j * 16, 16)
                        out_v[buf, be, t, sl] = (
                            out_v[buf, be, t, sl] * scales[i * 4 + j])
                return carry2

            lax.fori_loop(0, T, scale_body, 0)

        pltpu.async_copy(
            out_v.at[buf], out_hbm.at[pl.ds(g * 2, 2)], sem_o[buf])

    fire_gathers(0, 0)

    def step_body(s, carry):
        for buf in range(2):
            p = s * 2 + buf

            @pl.when(p + 1 < PAIRS_PER_W)
            def _():
                fire_gathers(p + 1, 1 - buf)

            wait_gathers(buf)

            @pl.when(p >= 2)
            def _():
                wait_out(buf)

            compute(wid * PAIRS_PER_W + p, buf)
        return carry

    lax.fori_loop(0, PAIRS_PER_W // 2, step_body, 0)
    wait_out(0)
    wait_out(1)


_sc_call = functools.partial(
    pl.kernel,
    out_type=jax.ShapeDtypeStruct((BATCH, T, OUT_D), jnp.float32),
    mesh=plsc.VectorSubcoreMesh(core_axis_name="c", subcore_axis_name="s"),
    compiler_params=pltpu.CompilerParams(use_tc_tiling_on_sc=False),
    scratch_types=[
        pltpu.VMEM((PAIRS_PER_W, IDX_PER_PAIR), jnp.int32),
        pltpu.VMEM((2, IDX_PER_PAIR, EMBED_DIM), jnp.float32),
        pltpu.VMEM((2, 2, T, OUT_D), jnp.float32),
        pltpu.SemaphoreType.DMA,
        pltpu.SemaphoreType.DMA,
        pltpu.SemaphoreType.DMA,
        pltpu.SemaphoreType.DMA,
    ],
)(_body)


def kernel(x, table):
    x2 = x.reshape(BATCH * T * NF // IDX_PER_PAIR, IDX_PER_PAIR)
    return _sc_call(x2, table)
